# baseline (speedup 1.0000x reference)
import functools

import jax
import jax.numpy as jnp
from jax import lax
from jax.experimental import pallas as pl
from jax.experimental.pallas import tpu as pltpu
from jax.experimental.pallas import tpu_sc as plsc

_B = 16384
_S = 200
_E = 3
_NW = 32
_IW = _B // _NW
_JC = 16
_NB = 12
_JT = _S - _NB * _JC


def _lookup_rows(idx_v, w_v, out_v, n_rows):
    def j_body(t, c2):
        for r in range(2):
            j = 2 * t + r
            for u in range(_IW // 16):
                o = u * 16
                v = idx_v[j, pl.ds(o, 16)]
                w0 = plsc.load_gather(w_v.at[0], [v])
                w1 = plsc.load_gather(w_v.at[1], [v])
                w2 = plsc.load_gather(w_v.at[2], [v])
                out_v[0, j, pl.ds(o, 16)] = w0
                out_v[1, j, pl.ds(o, 16)] = w1
                out_v[2, j, pl.ds(o, 16)] = w2
        return c2

    lax.fori_loop(0, n_rows // 2, j_body, 0)


def _emb_kernel(
    idx_hbm, w_hbm, out_hbm,
    w_v, idx_v0, idx_v1, out_v0, out_v1, si0, si1, so0, so1
):
    wid = lax.axis_index("s") * 2 + lax.axis_index("c")
    pltpu.sync_copy(w_hbm, w_v)
    i0 = wid * _IW
    idx_bufs = (idx_v0, idx_v1)
    idx_sems = (si0, si1)
    out_bufs = (out_v0, out_v1)
    out_sems = (so0, so1)

    def idx_src(b):
        return idx_hbm.at[pl.ds(b * _JC, _JC), pl.ds(i0, _IW)]

    def out_dst(b):
        return out_hbm.at[:, pl.ds(b * _JC, _JC), pl.ds(i0, _IW)]

    tail_idx_src = idx_hbm.at[pl.ds(_NB * _JC, _JT), pl.ds(i0, _IW)]
    tail_out_dst = out_hbm.at[:, pl.ds(_NB * _JC, _JT), pl.ds(i0, _IW)]

    def block(b, p):
        ib, isem = idx_bufs[p], idx_sems[p]
        ob, osem = out_bufs[p], out_sems[p]
        pltpu.make_async_copy(idx_src(b), ib, isem).wait()

        @pl.when(b + 1 < _NB)
        def _():
            pltpu.async_copy(
                idx_src(b + 1), idx_bufs[1 - p], idx_sems[1 - p]
            )

        @pl.when(b + 1 == _NB)
        def _():
            pltpu.async_copy(
                tail_idx_src,
                idx_bufs[1 - p].at[pl.ds(0, _JT), :],
                idx_sems[1 - p],
            )

        @pl.when(b >= 2)
        def _():
            pltpu.make_async_copy(ob, out_dst(b - 2), osem).wait()

        _lookup_rows(ib, w_v, ob, _JC)
        pltpu.async_copy(ob, out_dst(b), osem)

    pltpu.async_copy(idx_src(0), idx_bufs[0], idx_sems[0])

    def b_body(b, carry):
        @pl.when(b % 2 == 0)
        def _():
            block(b, 0)

        @pl.when(b % 2 == 1)
        def _():
            block(b, 1)

        return carry

    lax.fori_loop(0, _NB, b_body, 0)

    pltpu.make_async_copy(
        tail_idx_src, idx_bufs[0].at[pl.ds(0, _JT), :], idx_sems[0]
    ).wait()
    pltpu.make_async_copy(
        out_bufs[0], out_dst(_NB - 2), out_sems[0]
    ).wait()
    _lookup_rows(idx_bufs[0], w_v, out_bufs[0], _JT)
    pltpu.async_copy(
        out_bufs[0].at[:, pl.ds(0, _JT), :], tail_out_dst, out_sems[0]
    )
    pltpu.make_async_copy(
        out_bufs[1], out_dst(_NB - 1), out_sems[1]
    ).wait()
    pltpu.make_async_copy(
        out_bufs[0].at[:, pl.ds(0, _JT), :], tail_out_dst, out_sems[0]
    ).wait()


@jax.jit
def _emb(idx_t, wcols):
    mesh = plsc.VectorSubcoreMesh(core_axis_name="c", subcore_axis_name="s")
    run = functools.partial(
        pl.kernel,
        mesh=mesh,
        out_type=jax.ShapeDtypeStruct((_E, _S, _B), jnp.float32),
        scratch_types=[
            pltpu.VMEM((3, 16), jnp.float32),
            pltpu.VMEM((_JC, _IW), jnp.int32),
            pltpu.VMEM((_JC, _IW), jnp.int32),
            pltpu.VMEM((_E, _JC, _IW), jnp.float32),
            pltpu.VMEM((_E, _JC, _IW), jnp.float32),
            pltpu.SemaphoreType.DMA,
            pltpu.SemaphoreType.DMA,
            pltpu.SemaphoreType.DMA,
            pltpu.SemaphoreType.DMA,
        ],
        compiler_params=pltpu.CompilerParams(needs_layout_passes=False),
    )(_emb_kernel)
    return run(idx_t, wcols)


def kernel(arg, weight):
    wcols = jnp.pad(weight.T, ((0, 0), (0, 6)))
    out_t = _emb(arg.T.astype(jnp.int32), wcols)
    return jnp.transpose(out_t, (2, 1, 0))


# speedup vs baseline: 1.2154x; 1.2154x over previous
import functools

import jax
import jax.numpy as jnp
from jax import lax
from jax.experimental import pallas as pl
from jax.experimental.pallas import tpu as pltpu
from jax.experimental.pallas import tpu_sc as plsc

_B = 16384
_S = 200
_E = 3
_NW = 32
_IW = _B // _NW
_JC = 8
_NB = _S // _JC


def _lookup_rows(idx_v, w_v, out_v, n_rows):
    def j_body(t, c2):
        for r in range(2):
            j = 2 * t + r
            for u in range(_IW // 16):
                o = u * 16
                v = idx_v[j, pl.ds(o, 16)]
                w0 = plsc.load_gather(w_v.at[0], [v])
                w1 = plsc.load_gather(w_v.at[1], [v])
                w2 = plsc.load_gather(w_v.at[2], [v])
                out_v[0, j, pl.ds(o, 16)] = w0
                out_v[1, j, pl.ds(o, 16)] = w1
                out_v[2, j, pl.ds(o, 16)] = w2
        return c2

    lax.fori_loop(0, n_rows // 2, j_body, 0)


def _emb_kernel(
    idx_hbm, w_hbm, out_hbm,
    w_v, idx_v0, idx_v1, out_v0, out_v1, si0, si1, so0, so1
):
    wid = lax.axis_index("s") * 2 + lax.axis_index("c")
    pltpu.sync_copy(w_hbm, w_v)
    i0 = wid * _IW
    idx_bufs = (idx_v0, idx_v1)
    idx_sems = (si0, si1)
    out_bufs = (out_v0, out_v1)
    out_sems = (so0, so1)

    def idx_src(b):
        return idx_hbm.at[pl.ds(b * _JC, _JC), pl.ds(i0, _IW)]

    def out_dst(b):
        return out_hbm.at[:, pl.ds(b * _JC, _JC), pl.ds(i0, _IW)]

    def block(b, p):
        ib, isem = idx_bufs[p], idx_sems[p]
        ob, osem = out_bufs[p], out_sems[p]
        pltpu.make_async_copy(idx_src(b), ib, isem).wait()

        @pl.when(b + 1 < _NB)
        def _():
            pltpu.async_copy(
                idx_src(b + 1), idx_bufs[1 - p], idx_sems[1 - p]
            )

        @pl.when(b >= 2)
        def _():
            pltpu.make_async_copy(ob, out_dst(b - 2), osem).wait()

        _lookup_rows(ib, w_v, ob, _JC)
        pltpu.async_copy(ob, out_dst(b), osem)

    pltpu.async_copy(idx_src(0), idx_bufs[0], idx_sems[0])

    def b_body(b, carry):
        @pl.when(b % 2 == 0)
        def _():
            block(b, 0)

        @pl.when(b % 2 == 1)
        def _():
            block(b, 1)

        return carry

    lax.fori_loop(0, _NB, b_body, 0)
    pltpu.make_async_copy(
        out_bufs[1], out_dst(_NB - 2), out_sems[1]
    ).wait()
    pltpu.make_async_copy(
        out_bufs[0], out_dst(_NB - 1), out_sems[0]
    ).wait()


@jax.jit
def _emb(idx_t, wcols):
    mesh = plsc.VectorSubcoreMesh(core_axis_name="c", subcore_axis_name="s")
    run = functools.partial(
        pl.kernel,
        mesh=mesh,
        out_type=jax.ShapeDtypeStruct((_E, _S, _B), jnp.float32),
        scratch_types=[
            pltpu.VMEM((3, 16), jnp.float32),
            pltpu.VMEM((_JC, _IW), jnp.int32),
            pltpu.VMEM((_JC, _IW), jnp.int32),
            pltpu.VMEM((_E, _JC, _IW), jnp.float32),
            pltpu.VMEM((_E, _JC, _IW), jnp.float32),
            pltpu.SemaphoreType.DMA,
            pltpu.SemaphoreType.DMA,
            pltpu.SemaphoreType.DMA,
            pltpu.SemaphoreType.DMA,
        ],
        compiler_params=pltpu.CompilerParams(needs_layout_passes=False),
    )(_emb_kernel)
    return run(idx_t, wcols)


def kernel(arg, weight):
    wcols = jnp.pad(weight.T, ((0, 0), (0, 6)))
    out_t = _emb(arg.T.astype(jnp.int32), wcols)
    return jnp.transpose(out_t, (2, 1, 0))
